# baseline (speedup 1.0000x reference)
import functools

import jax
import jax.numpy as jnp
from jax import lax
from jax.experimental import pallas as pl
from jax.experimental.pallas import tpu as pltpu
from jax.experimental.pallas import tpu_sc as plsc

_K = 16
_TQ = 256
_NC = 2
_NS = 16
_CH = 1024


def _topk_body(pc_ref, q_ref, idx_ref):
    pc = pc_ref[0]
    q = q_ref[0]
    n = pc.shape[1]
    norms = jnp.sum(pc * pc, axis=0)[None, :]
    qn = jnp.sum(q * q, axis=0)[:, None]
    inner = lax.dot_general(q, pc, (((0,), (0,)), ((), ())),
                            preferred_element_type=jnp.float32)
    dist = qn + norms - 2.0 * inner
    iota = lax.broadcasted_iota(jnp.int32, dist.shape, 1)
    big = jnp.int32(n)
    for i in range(_K + 1):
        m = jnp.min(dist, axis=1, keepdims=True)
        amin = jnp.min(jnp.where(dist == m, iota, big), axis=1)
        idx_ref[0, i, :] = amin
        if i < _K:
            dist = jnp.where(iota == amin[:, None], jnp.inf, dist)


def _topk_call(pc):
    b, dims, n = pc.shape
    grid = (b, n // _TQ)
    return pl.pallas_call(
        _topk_body,
        grid=grid,
        in_specs=[
            pl.BlockSpec((1, dims, n), lambda bi, ti: (bi, 0, 0)),
            pl.BlockSpec((1, dims, _TQ), lambda bi, ti: (bi, 0, ti)),
        ],
        out_specs=pl.BlockSpec((1, 24, _TQ), lambda bi, ti: (bi, 0, ti)),
        out_shape=jax.ShapeDtypeStruct((b, 24, n), jnp.int32),
    )(pc, pc)


def _gather_body(pc_hbm, idx_hbm, out_hbm, table_v, idx_sp, cent_f, diff_f):
    b, dims, n = pc_hbm.shape
    wid = lax.axis_index("s") * _NC + lax.axis_index("c")
    groups = _NC * _NS // b
    bi = wid // groups
    dg = wid % groups
    dper = dims // groups
    pltpu.sync_copy(idx_hbm.at[bi, pl.ds(0, (_K + 1) * n)], idx_sp)
    kio = (lax.iota(jnp.int32, 16) + 1) * n

    def run_task(t, _):
        di = dg * dper + t
        pltpu.sync_copy(pc_hbm.at[bi, di], table_v)

        def run_chunk(ci, _):
            n0 = ci * _CH

            def run_group(j, _):
                gbase = n0 + j * 16
                ct = table_v[pl.ds(gbase, 16)]
                for r in range(16):
                    iv = plsc.load_gather(
                        idx_sp, [kio + jnp.broadcast_to(gbase + r, (16,))])
                    g = plsc.load_gather(table_v, [iv])
                    c = jnp.broadcast_to(ct[r], (16,))
                    o = (j * 16 + r) * _K
                    diff_f[pl.ds(o, 16)] = g - c
                    cent_f[pl.ds(o, 16)] = c
                return ()

            lax.fori_loop(0, _CH // 16, run_group, ())
            pltpu.sync_copy(cent_f, out_hbm.at[bi, di, pl.ds(n0 * _K, _CH * _K)])
            pltpu.sync_copy(diff_f,
                            out_hbm.at[bi, dims + di, pl.ds(n0 * _K, _CH * _K)])
            return ()

        lax.fori_loop(0, n // _CH, run_chunk, ())
        return ()

    lax.fori_loop(0, dper, run_task, ())


def _gather_call(pc, idx_all):
    b, dims, n = pc.shape
    mesh = plsc.VectorSubcoreMesh(core_axis_name="c", subcore_axis_name="s")
    kern = functools.partial(
        pl.kernel,
        out_type=jax.ShapeDtypeStruct((b, 2 * dims, n * _K), jnp.float32),
        mesh=mesh,
        compiler_params=pltpu.CompilerParams(needs_layout_passes=False),
        scratch_types=[
            pltpu.VMEM((n,), jnp.float32),
            pltpu.VMEM(((_K + 1) * n,), jnp.int32),
            pltpu.VMEM((_CH * _K,), jnp.float32),
            pltpu.VMEM((_CH * _K,), jnp.float32),
        ],
    )(_gather_body)
    return kern(pc, idx_all.reshape(b, 24 * n))


def kernel(point_cloud):
    idx_all = _topk_call(point_cloud)
    edge = _gather_call(point_cloud, idx_all)
    idx = idx_all[:, 1:_K + 1, :]
    b, dims, n = point_cloud.shape
    return (edge.reshape(b, 2 * dims, n, _K), idx)


# speedup vs baseline: 1.3515x; 1.3515x over previous
import functools

import jax
import jax.numpy as jnp
from jax import lax
from jax.experimental import pallas as pl
from jax.experimental.pallas import tpu as pltpu
from jax.experimental.pallas import tpu_sc as plsc

_K = 16
_TQ = 256
_NC = 2
_NS = 16
_CH = 1024


def _topk_body(pc_ref, q_ref, idx_ref):
    pc = pc_ref[0]
    q = q_ref[0]
    n = pc.shape[1]
    norms = jnp.sum(pc * pc, axis=0)[None, :]
    qn = jnp.sum(q * q, axis=0)[:, None]
    inner = lax.dot_general(q, pc, (((0,), (0,)), ((), ())),
                            preferred_element_type=jnp.float32)
    dist = qn + norms - 2.0 * inner
    iota = lax.broadcasted_iota(jnp.int32, dist.shape, 1)
    big = jnp.int32(n)
    for i in range(_K + 1):
        m = jnp.min(dist, axis=1, keepdims=True)
        amin = jnp.min(jnp.where(dist == m, iota, big), axis=1)
        idx_ref[0, i, :] = amin
        if i < _K:
            dist = jnp.where(iota == amin[:, None], jnp.inf, dist)


def _topk_call(pc):
    b, dims, n = pc.shape
    grid = (b, n // _TQ)
    return pl.pallas_call(
        _topk_body,
        grid=grid,
        in_specs=[
            pl.BlockSpec((1, dims, n), lambda bi, ti: (bi, 0, 0)),
            pl.BlockSpec((1, dims, _TQ), lambda bi, ti: (bi, 0, ti)),
        ],
        out_specs=pl.BlockSpec((1, 24, _TQ), lambda bi, ti: (bi, 0, ti)),
        out_shape=jax.ShapeDtypeStruct((b, 24, n), jnp.int32),
    )(pc, pc)


def _gather_body(pc_hbm, idx_hbm, out_hbm, table_v, idx_sp, cent_f, diff_f):
    b, dims, n = pc_hbm.shape
    wid = lax.axis_index("s") * _NC + lax.axis_index("c")
    groups = _NC * _NS // b
    bi = wid // groups
    dg = wid % groups
    dper = dims // groups
    pltpu.sync_copy(idx_hbm.at[bi], idx_sp)

    def run_task(t, _):
        di = dg * dper + t
        pltpu.sync_copy(pc_hbm.at[bi, di], table_v)

        def run_chunk(ci, _):
            n0 = ci * _CH

            @plsc.parallel_loop(0, _CH, step=16, unroll=2)
            def run_group(j):
                ct = table_v[pl.ds(n0 + j, 16)]
                for r in range(16):
                    iv = idx_sp[pl.ds((n0 + j + r) * _K, 16)]
                    g = plsc.load_gather(table_v, [iv])
                    c = jnp.broadcast_to(ct[r], (16,))
                    o = (j + r) * _K
                    diff_f[pl.ds(o, 16)] = g - c
                    cent_f[pl.ds(o, 16)] = c

            pltpu.sync_copy(cent_f, out_hbm.at[bi, di, pl.ds(n0 * _K, _CH * _K)])
            pltpu.sync_copy(diff_f,
                            out_hbm.at[bi, dims + di, pl.ds(n0 * _K, _CH * _K)])
            return ()

        lax.fori_loop(0, n // _CH, run_chunk, ())
        return ()

    lax.fori_loop(0, dper, run_task, ())


def _gather_call(pc, idx_all):
    b, dims, n = pc.shape
    mesh = plsc.VectorSubcoreMesh(core_axis_name="c", subcore_axis_name="s")
    kern = functools.partial(
        pl.kernel,
        out_type=jax.ShapeDtypeStruct((b, 2 * dims, n * _K), jnp.float32),
        mesh=mesh,
        compiler_params=pltpu.CompilerParams(needs_layout_passes=False),
        scratch_types=[
            pltpu.VMEM((n,), jnp.float32),
            pltpu.VMEM((n * _K,), jnp.int32),
            pltpu.VMEM((_CH * _K,), jnp.float32),
            pltpu.VMEM((_CH * _K,), jnp.float32),
        ],
    )(_gather_body)
    idx_t = jnp.transpose(idx_all[:, 1:_K + 1, :], (0, 2, 1))
    return kern(pc, idx_t.reshape(b, n * _K))


def kernel(point_cloud):
    idx_all = _topk_call(point_cloud)
    edge = _gather_call(point_cloud, idx_all)
    idx = idx_all[:, 1:_K + 1, :]
    b, dims, n = point_cloud.shape
    return (edge.reshape(b, 2 * dims, n, _K), idx)


# speedup vs baseline: 1.4756x; 1.0919x over previous
import functools

import jax
import jax.numpy as jnp
from jax import lax
from jax.experimental import pallas as pl
from jax.experimental.pallas import tpu as pltpu
from jax.experimental.pallas import tpu_sc as plsc

_K = 16
_TQ = 256
_NC = 2
_NS = 16
_CH = 1024


def _topk_body(pc_ref, q_ref, idx_ref):
    pc = pc_ref[0]
    q = q_ref[0]
    n = pc.shape[1]
    norms = jnp.sum(pc * pc, axis=0)[None, :]
    qn = jnp.sum(q * q, axis=0)[:, None]
    inner = lax.dot_general(q, pc, (((0,), (0,)), ((), ())),
                            preferred_element_type=jnp.float32)
    dist = qn + norms - 2.0 * inner
    iota = lax.broadcasted_iota(jnp.int32, dist.shape, 1)
    for i in range(_K + 1):
        amin = jnp.argmin(dist, axis=1).astype(jnp.int32)
        idx_ref[0, i, :] = amin
        if i < _K:
            dist = jnp.where(iota == amin[:, None], jnp.inf, dist)


def _topk_call(pc):
    b, dims, n = pc.shape
    grid = (b, n // _TQ)
    return pl.pallas_call(
        _topk_body,
        grid=grid,
        in_specs=[
            pl.BlockSpec((1, dims, n), lambda bi, ti: (bi, 0, 0)),
            pl.BlockSpec((1, dims, _TQ), lambda bi, ti: (bi, 0, ti)),
        ],
        out_specs=pl.BlockSpec((1, 24, _TQ), lambda bi, ti: (bi, 0, ti)),
        out_shape=jax.ShapeDtypeStruct((b, 24, n), jnp.int32),
    )(pc, pc)


def _gather_body(pc_hbm, idx_hbm, out_hbm, table_v, idx_sp, cent_f, diff_f):
    b, dims, n = pc_hbm.shape
    wid = lax.axis_index("s") * _NC + lax.axis_index("c")
    groups = _NC * _NS // b
    bi = wid // groups
    dg = wid % groups
    dper = dims // groups
    pltpu.sync_copy(idx_hbm.at[bi], idx_sp)

    def run_task(t, _):
        di = dg * dper + t
        pltpu.sync_copy(pc_hbm.at[bi, di], table_v)

        def run_chunk(ci, _):
            n0 = ci * _CH

            @plsc.parallel_loop(0, _CH, step=16, unroll=2)
            def run_group(j):
                ct = table_v[pl.ds(n0 + j, 16)]
                for r in range(16):
                    iv = idx_sp[pl.ds((n0 + j + r) * _K, 16)]
                    g = plsc.load_gather(table_v, [iv])
                    c = jnp.broadcast_to(ct[r], (16,))
                    o = (j + r) * _K
                    diff_f[pl.ds(o, 16)] = g - c
                    cent_f[pl.ds(o, 16)] = c

            pltpu.sync_copy(cent_f, out_hbm.at[bi, di, pl.ds(n0 * _K, _CH * _K)])
            pltpu.sync_copy(diff_f,
                            out_hbm.at[bi, dims + di, pl.ds(n0 * _K, _CH * _K)])
            return ()

        lax.fori_loop(0, n // _CH, run_chunk, ())
        return ()

    lax.fori_loop(0, dper, run_task, ())


def _gather_call(pc, idx_all):
    b, dims, n = pc.shape
    mesh = plsc.VectorSubcoreMesh(core_axis_name="c", subcore_axis_name="s")
    kern = functools.partial(
        pl.kernel,
        out_type=jax.ShapeDtypeStruct((b, 2 * dims, n * _K), jnp.float32),
        mesh=mesh,
        compiler_params=pltpu.CompilerParams(needs_layout_passes=False),
        scratch_types=[
            pltpu.VMEM((n,), jnp.float32),
            pltpu.VMEM((n * _K,), jnp.int32),
            pltpu.VMEM((_CH * _K,), jnp.float32),
            pltpu.VMEM((_CH * _K,), jnp.float32),
        ],
    )(_gather_body)
    idx_t = jnp.transpose(idx_all[:, 1:_K + 1, :], (0, 2, 1))
    return kern(pc, idx_t.reshape(b, n * _K))


def kernel(point_cloud):
    idx_all = _topk_call(point_cloud)
    edge = _gather_call(point_cloud, idx_all)
    idx = idx_all[:, 1:_K + 1, :]
    b, dims, n = point_cloud.shape
    return (edge.reshape(b, 2 * dims, n, _K), idx)
